# baseline (speedup 1.0000x reference)
import functools
import jax
import jax.numpy as jnp
from jax import lax
from jax.experimental import pallas as pl
from jax.experimental.pallas import tpu as pltpu
from jax.experimental.pallas import tpu_sc as plsc

D = 64
NC, NS = 2, 16
NW = NC * NS
S = 128
K = 8
CHUNK = S * K

_mesh = plsc.VectorSubcoreMesh(core_axis_name="c", subcore_axis_name="s")


def _make_gather(n_total):
    per_w = n_total // NW
    n_chunks = per_w // CHUNK

    @functools.partial(
        pl.kernel,
        mesh=_mesh,
        out_type=jax.ShapeDtypeStruct((n_total, D), jnp.float32),
        scratch_types=[
            pltpu.VMEM((K, S), jnp.int32),
            pltpu.VMEM((CHUNK, D), jnp.float32),
            pltpu.SemaphoreType.DMA,
        ],
        compiler_params=pltpu.CompilerParams(use_tc_tiling_on_sc=False),
    )
    def gather_kernel(idx_hbm, table_hbm, out_hbm, idx_v, rows_v, sem):
        wid = lax.axis_index("s") * NC + lax.axis_index("c")
        base_w = wid * n_chunks

        def body(i, carry):
            chunk_row = base_w + i
            pltpu.sync_copy(idx_hbm.at[chunk_row], idx_v)
            copies = [
                pltpu.async_copy(
                    table_hbm.at[idx_v.at[j]],
                    rows_v.at[pl.ds(j * S, S)],
                    sem,
                )
                for j in range(K)
            ]
            for cp in copies:
                cp.wait()
            pltpu.sync_copy(rows_v, out_hbm.at[pl.ds(chunk_row * CHUNK, CHUNK)])
            return carry

        lax.fori_loop(0, n_chunks, body, 0)

    return gather_kernel


_gather = _make_gather(4096 * 200)


def kernel(batch_data, table):
    idx = batch_data.reshape(-1, K, S).astype(jnp.int32)
    out = _gather(idx, table)
    return out.reshape(batch_data.shape + (D,))


# speedup vs baseline: 1.0103x; 1.0103x over previous
import functools
import jax
import jax.numpy as jnp
from jax import lax
from jax.experimental import pallas as pl
from jax.experimental.pallas import tpu as pltpu
from jax.experimental.pallas import tpu_sc as plsc

D = 64
NC, NS = 2, 16
NW = NC * NS
S = 128
K = 5
CHUNK = S * K

_mesh = plsc.VectorSubcoreMesh(core_axis_name="c", subcore_axis_name="s")


def _make_gather(n_total):
    per_w = n_total // NW
    n_chunks = per_w // CHUNK
    n_pairs = n_chunks // 2

    @functools.partial(
        pl.kernel,
        mesh=_mesh,
        out_type=jax.ShapeDtypeStruct((n_total, D), jnp.float32),
        scratch_types=[
            pltpu.VMEM((2, K, S), jnp.int32),
            pltpu.VMEM((CHUNK, D), jnp.float32),
            pltpu.VMEM((CHUNK, D), jnp.float32),
            pltpu.SemaphoreType.DMA,
            pltpu.SemaphoreType.DMA,
            pltpu.SemaphoreType.DMA,
            pltpu.SemaphoreType.DMA,
        ],
        compiler_params=pltpu.CompilerParams(use_tc_tiling_on_sc=False),
    )
    def gather_kernel(idx_hbm, table_hbm, out_hbm, idx_v,
                      rows0, rows1, sg0, sg1, so0, so1):
        rows = [rows0, rows1]
        sg = [sg0, sg1]
        so = [so0, so1]
        wid = lax.axis_index("s") * NC + lax.axis_index("c")
        chunk0 = wid * n_chunks

        def fire_gather(ci, p):
            pltpu.sync_copy(idx_hbm.at[chunk0 + ci], idx_v.at[p])
            for j in range(K):
                pltpu.async_copy(
                    table_hbm.at[idx_v.at[p, j]],
                    rows[p].at[pl.ds(j * S, S)],
                    sg[p],
                )

        def drain_gather(p):
            pltpu.make_async_copy(
                out_hbm.at[pl.ds(0, CHUNK)], rows[p], sg[p]
            ).wait()

        def fire_out(ci, p):
            pltpu.async_copy(
                rows[p], out_hbm.at[pl.ds((chunk0 + ci) * CHUNK, CHUNK)], so[p]
            )

        def drain_out(p):
            pltpu.make_async_copy(
                rows[p], out_hbm.at[pl.ds(0, CHUNK)], so[p]
            ).wait()

        fire_gather(0, 0)

        def body(m, carry):
            ci = 2 * m

            @pl.when(m > 0)
            def _():
                drain_out(1)

            fire_gather(ci + 1, 1)
            drain_gather(0)
            fire_out(ci, 0)
            drain_out(0)

            @pl.when(m < n_pairs - 1)
            def _():
                fire_gather(ci + 2, 0)

            drain_gather(1)
            fire_out(ci + 1, 1)
            return carry

        lax.fori_loop(0, n_pairs, body, 0)
        drain_out(1)

    return gather_kernel


_gather = _make_gather(4096 * 200)


def kernel(batch_data, table):
    idx = batch_data.reshape(-1, K, S).astype(jnp.int32)
    out = _gather(idx, table)
    return out.reshape(batch_data.shape + (D,))
